# baseline (speedup 1.0000x reference)
import functools

import numpy as np
import jax
import jax.numpy as jnp
from jax import lax
from jax.experimental import pallas as pl
from jax.experimental.pallas import tpu as pltpu
from jax.experimental.pallas import tpu_sc as plsc

N_LEVELS = 12
N_FEATURES = 4
TABLE_SIZE = 524288
BASE_RES = 16
MAX_RES = 2048
FEATURE_DIM = N_LEVELS * N_FEATURES
OUT_DIM = FEATURE_DIM + 1
N_POINTS = 262144

P2 = int(np.uint32(2654435761).view(np.int32))
P3 = 805459861

NC = 2
NS = 16
L = 16


def _resolutions(n_levels=N_LEVELS):
    b = np.exp((np.log(MAX_RES) - np.log(BASE_RES)) / (N_LEVELS - 1))
    return [int(np.floor(BASE_RES * (b ** l))) for l in range(n_levels)]


def _build_encode(n_points, n_levels, table_size, chunk):
    nw = NC * NS
    pts_per_w = n_points // nw
    assert pts_per_w % chunk == 0
    n_chunks = pts_per_w // chunk
    groups = chunk // L
    rows_n = 8 * chunk
    mask = table_size - 1
    res_list = _resolutions(n_levels)
    feat_dim = n_levels * N_FEATURES

    mesh = plsc.VectorSubcoreMesh(core_axis_name="c", subcore_axis_name="s",
                                  num_cores=NC, num_subcores=NS)

    assert n_chunks % 2 == 0

    @functools.partial(
        pl.kernel,
        out_type=jax.ShapeDtypeStruct((feat_dim, n_points), jnp.float32),
        mesh=mesh,
        scratch_types=(
            [pltpu.VMEM((chunk,), jnp.float32) for _ in range(3)]
            + [pltpu.VMEM((rows_n,), jnp.int32)
               for _ in range(2 * n_levels)]
            + [pltpu.VMEM((2, 3 * n_levels, chunk), jnp.float32)]
            + [pltpu.VMEM((rows_n,), jnp.float32)
               for _ in range(4)]
            + [pltpu.VMEM((feat_dim, chunk), jnp.float32)]
            + [pltpu.SemaphoreType.DMA((2,))]
        ),
    )
    def encode(xs_hbm, ys_hbm, zs_hbm, pa_hbm, pb_hbm, feats_hbm, *refs):
        xyz_refs = refs[0:3]
        idx_refs = [refs[3:3 + n_levels],
                    refs[3 + n_levels:3 + 2 * n_levels]]
        frac_v = refs[3 + 2 * n_levels]
        pair_refs = refs[4 + 2 * n_levels:8 + 2 * n_levels]
        feats_v = refs[8 + 2 * n_levels]
        sems = refs[9 + 2 * n_levels]

        wid = lax.axis_index("s") * NC + lax.axis_index("c")
        planes = (pa_hbm, pb_hbm)

        def fire(l, b, ph):
            for j in range(2):
                pltpu.make_async_copy(
                    planes[j].at[idx_refs[b][l]], pair_refs[2 * ph + j],
                    sems.at[ph]).start()

        def wait(l, b, ph):
            for j in range(2):
                pltpu.make_async_copy(
                    planes[j].at[idx_refs[b][l]], pair_refs[2 * ph + j],
                    sems.at[ph]).wait()

        def phase_a(ci, b):
            @pl.when(ci < n_chunks)
            def _():
                base = wid * pts_per_w + ci * chunk
                for d, src in enumerate((xs_hbm, ys_hbm, zs_hbm)):
                    pltpu.sync_copy(src.at[pl.ds(base, chunk)], xyz_refs[d])

                def group_a(g, _):
                    off = g * L
                    x = xyz_refs[0][pl.ds(off, L)]
                    y = xyz_refs[1][pl.ds(off, L)]
                    z = xyz_refs[2][pl.ds(off, L)]
                    for l in range(n_levels):
                        res = float(res_list[l])
                        sx = x * res
                        sy = y * res
                        sz = z * res
                        ix = sx.astype(jnp.int32)
                        iy = sy.astype(jnp.int32)
                        iz = sz.astype(jnp.int32)
                        frac_v[b, 3 * l + 0, pl.ds(off, L)] = (
                            sx - ix.astype(jnp.float32))
                        frac_v[b, 3 * l + 1, pl.ds(off, L)] = (
                            sy - iy.astype(jnp.float32))
                        frac_v[b, 3 * l + 2, pl.ds(off, L)] = (
                            sz - iz.astype(jnp.float32))
                        hy0 = iy * P2
                        hz0 = iz * P3
                        hx1 = ix + 1
                        hy1 = hy0 + P2
                        hz1 = hz0 + P3
                        lbase = l * table_size
                        for corner in range(8):
                            hx = hx1 if (corner & 1) else ix
                            hy = hy1 if (corner & 2) else hy0
                            hz = hz1 if (corner & 4) else hz0
                            h = ((hx ^ hy ^ hz) & mask) + lbase
                            pos = corner * chunk + off
                            idx_refs[b][l][pl.ds(pos, L)] = h
                    return 0

                lax.fori_loop(0, groups, group_a, 0, unroll=False)

        def run_chunk(ci, b):
            fire(0, b, 0)
            fire(1, b, 1)
            phase_a(ci + 1, 1 - b)
            for l in range(n_levels):
                ph = l % 2
                wait(l, b, ph)
                pa = pair_refs[2 * ph]
                pb = pair_refs[2 * ph + 1]

                def group_c(g, _, l=l, pa=pa, pb=pb, b=b):
                    off = g * L
                    fx = frac_v[b, 3 * l + 0, pl.ds(off, L)]
                    fy = frac_v[b, 3 * l + 1, pl.ds(off, L)]
                    fz = frac_v[b, 3 * l + 2, pl.ds(off, L)]
                    gx = 1.0 - fx
                    gy = 1.0 - fy
                    gz = 1.0 - fz
                    txy = (gx * gy, fx * gy, gx * fy, fx * fy)
                    acc = [jnp.zeros((L,), jnp.float32) for _ in range(4)]
                    himask = jnp.full((L,), -65536, jnp.int32)
                    for corner in range(8):
                        pos = corner * chunk + off
                        wv = txy[corner & 3] * (fz if (corner & 4) else gz)
                        for j, plane in ((0, pa), (2, pb)):
                            raw = lax.bitcast_convert_type(
                                plane[pl.ds(pos, L)], jnp.int32)
                            flo = lax.bitcast_convert_type(
                                lax.shift_left(raw, 16), jnp.float32)
                            fhi = lax.bitcast_convert_type(
                                lax.bitwise_and(raw, himask), jnp.float32)
                            acc[j] = acc[j] + wv * flo
                            acc[j + 1] = acc[j + 1] + wv * fhi
                    for j in range(4):
                        feats_v[N_FEATURES * l + j, pl.ds(off, L)] = acc[j]
                    return 0

                lax.fori_loop(0, groups, group_c, 0, unroll=False)
                if l + 2 < n_levels:
                    fire(l + 2, b, ph)
            base = wid * pts_per_w + ci * chunk
            pltpu.sync_copy(feats_v, feats_hbm.at[:, pl.ds(base, chunk)])

        phase_a(jnp.int32(0), 0)

        def pair_body(cp, _):
            run_chunk(2 * cp, 0)
            run_chunk(2 * cp + 1, 1)
            return 0

        lax.fori_loop(0, n_chunks // 2, pair_body, 0, unroll=False)

    return encode


def _matmul(feats_t, w, block_n=2048):
    n = feats_t.shape[1]
    fd, od = w.shape

    def mm_kernel(f_ref, w_ref, o_ref):
        o_ref[...] = lax.dot_general(
            f_ref[...], w_ref[...], (((0,), (0,)), ((), ())),
            preferred_element_type=jnp.float32)

    return pl.pallas_call(
        mm_kernel,
        grid=(n // block_n,),
        in_specs=[
            pl.BlockSpec((fd, block_n), lambda i: (0, i)),
            pl.BlockSpec((fd, od), lambda i: (0, 0)),
        ],
        out_specs=pl.BlockSpec((block_n, od), lambda i: (i, 0)),
        out_shape=jax.ShapeDtypeStruct((n, od), jnp.float32),
    )(feats_t, w)


_encode_full = _build_encode(N_POINTS, N_LEVELS, TABLE_SIZE, chunk=256)


def kernel(inputs, tables, W):
    xyz = inputs.T
    t = jnp.transpose(tables, (2, 0, 1)).reshape(N_FEATURES, -1)
    xs, ys, zs = xyz[0], xyz[1], xyz[2]
    bits = lax.bitcast_convert_type(t, jnp.int32)
    rne = bits + 0x7FFF + (lax.shift_right_logical(bits, 16) & 1)
    himask = jnp.int32(-65536)

    def pack(lo, hi):
        word = lax.shift_right_logical(rne[lo], 16) | (rne[hi] & himask)
        return lax.bitcast_convert_type(word, jnp.float32)

    feats_t = _encode_full(xs, ys, zs, pack(0, 1), pack(2, 3))
    return _matmul(feats_t, W)


# speedup vs baseline: 1.0098x; 1.0098x over previous
import functools

import numpy as np
import jax
import jax.numpy as jnp
from jax import lax
from jax.experimental import pallas as pl
from jax.experimental.pallas import tpu as pltpu
from jax.experimental.pallas import tpu_sc as plsc

N_LEVELS = 12
N_FEATURES = 4
TABLE_SIZE = 524288
BASE_RES = 16
MAX_RES = 2048
FEATURE_DIM = N_LEVELS * N_FEATURES
OUT_DIM = FEATURE_DIM + 1
N_POINTS = 262144

P2 = int(np.uint32(2654435761).view(np.int32))
P3 = 805459861

NC = 2
NS = 16
L = 16


def _resolutions(n_levels=N_LEVELS):
    b = np.exp((np.log(MAX_RES) - np.log(BASE_RES)) / (N_LEVELS - 1))
    return [int(np.floor(BASE_RES * (b ** l))) for l in range(n_levels)]


def _build_encode(n_points, n_levels, table_size, chunk):
    nw = NC * NS
    pts_per_w = n_points // nw
    assert pts_per_w % chunk == 0
    n_chunks = pts_per_w // chunk
    groups = chunk // L
    rows_n = 8 * chunk
    mask = table_size - 1
    res_list = _resolutions(n_levels)
    feat_dim = n_levels * N_FEATURES

    mesh = plsc.VectorSubcoreMesh(core_axis_name="c", subcore_axis_name="s",
                                  num_cores=NC, num_subcores=NS)

    assert n_chunks % 2 == 0

    @functools.partial(
        pl.kernel,
        out_type=jax.ShapeDtypeStruct((feat_dim, n_points), jnp.float32),
        mesh=mesh,
        scratch_types=(
            [pltpu.VMEM((3, chunk), jnp.float32)]
            + [pltpu.VMEM((rows_n,), jnp.int32)
               for _ in range(2 * n_levels)]
            + [pltpu.VMEM((2, 3 * n_levels, chunk), jnp.float32)]
            + [pltpu.VMEM((rows_n,), jnp.float32)
               for _ in range(4)]
            + [pltpu.VMEM((feat_dim, chunk), jnp.float32)]
            + [pltpu.SemaphoreType.DMA((2,))]
        ),
    )
    def encode(xyz_hbm, pa_hbm, pb_hbm, feats_hbm, *refs):
        xyz_v = refs[0]
        idx_refs = [refs[1:1 + n_levels],
                    refs[1 + n_levels:1 + 2 * n_levels]]
        frac_v = refs[1 + 2 * n_levels]
        pair_refs = refs[2 + 2 * n_levels:6 + 2 * n_levels]
        feats_v = refs[6 + 2 * n_levels]
        sems = refs[7 + 2 * n_levels]

        wid = lax.axis_index("s") * NC + lax.axis_index("c")
        planes = (pa_hbm, pb_hbm)

        def fire(l, b, ph):
            for j in range(2):
                pltpu.make_async_copy(
                    planes[j].at[idx_refs[b][l]], pair_refs[2 * ph + j],
                    sems.at[ph]).start()

        def wait(l, b, ph):
            for j in range(2):
                pltpu.make_async_copy(
                    planes[j].at[idx_refs[b][l]], pair_refs[2 * ph + j],
                    sems.at[ph]).wait()

        def phase_a(ci, b):
            @pl.when(ci < n_chunks)
            def _():
                base = wid * pts_per_w + ci * chunk
                pltpu.sync_copy(xyz_hbm.at[:, pl.ds(base, chunk)], xyz_v)

                def group_a(g, _):
                    off = g * L
                    x = xyz_v[0, pl.ds(off, L)]
                    y = xyz_v[1, pl.ds(off, L)]
                    z = xyz_v[2, pl.ds(off, L)]
                    for l in range(n_levels):
                        res = float(res_list[l])
                        sx = x * res
                        sy = y * res
                        sz = z * res
                        ix = sx.astype(jnp.int32)
                        iy = sy.astype(jnp.int32)
                        iz = sz.astype(jnp.int32)
                        frac_v[b, 3 * l + 0, pl.ds(off, L)] = (
                            sx - ix.astype(jnp.float32))
                        frac_v[b, 3 * l + 1, pl.ds(off, L)] = (
                            sy - iy.astype(jnp.float32))
                        frac_v[b, 3 * l + 2, pl.ds(off, L)] = (
                            sz - iz.astype(jnp.float32))
                        hy0 = iy * P2
                        hz0 = iz * P3
                        hx1 = ix + 1
                        hy1 = hy0 + P2
                        hz1 = hz0 + P3
                        lbase = l * table_size
                        for corner in range(8):
                            hx = hx1 if (corner & 1) else ix
                            hy = hy1 if (corner & 2) else hy0
                            hz = hz1 if (corner & 4) else hz0
                            h = ((hx ^ hy ^ hz) & mask) + lbase
                            pos = corner * chunk + off
                            idx_refs[b][l][pl.ds(pos, L)] = h
                    return 0

                lax.fori_loop(0, groups, group_a, 0, unroll=False)

        def run_chunk(ci, b):
            fire(0, b, 0)
            fire(1, b, 1)
            phase_a(ci + 1, 1 - b)
            for l in range(n_levels):
                ph = l % 2
                wait(l, b, ph)
                pa = pair_refs[2 * ph]
                pb = pair_refs[2 * ph + 1]

                def group_c(g, _, l=l, pa=pa, pb=pb, b=b):
                    off = g * L
                    fx = frac_v[b, 3 * l + 0, pl.ds(off, L)]
                    fy = frac_v[b, 3 * l + 1, pl.ds(off, L)]
                    fz = frac_v[b, 3 * l + 2, pl.ds(off, L)]
                    gx = 1.0 - fx
                    gy = 1.0 - fy
                    gz = 1.0 - fz
                    txy = (gx * gy, fx * gy, gx * fy, fx * fy)
                    acc = [jnp.zeros((L,), jnp.float32) for _ in range(4)]
                    himask = jnp.full((L,), -65536, jnp.int32)
                    for corner in range(8):
                        pos = corner * chunk + off
                        wv = txy[corner & 3] * (fz if (corner & 4) else gz)
                        for j, plane in ((0, pa), (2, pb)):
                            raw = lax.bitcast_convert_type(
                                plane[pl.ds(pos, L)], jnp.int32)
                            flo = lax.bitcast_convert_type(
                                lax.shift_left(raw, 16), jnp.float32)
                            fhi = lax.bitcast_convert_type(
                                lax.bitwise_and(raw, himask), jnp.float32)
                            acc[j] = acc[j] + wv * flo
                            acc[j + 1] = acc[j + 1] + wv * fhi
                    for j in range(4):
                        feats_v[N_FEATURES * l + j, pl.ds(off, L)] = acc[j]
                    return 0

                lax.fori_loop(0, groups, group_c, 0, unroll=False)
                if l + 2 < n_levels:
                    fire(l + 2, b, ph)
            base = wid * pts_per_w + ci * chunk
            pltpu.sync_copy(feats_v, feats_hbm.at[:, pl.ds(base, chunk)])

        phase_a(jnp.int32(0), 0)

        def pair_body(cp, _):
            run_chunk(2 * cp, 0)
            run_chunk(2 * cp + 1, 1)
            return 0

        lax.fori_loop(0, n_chunks // 2, pair_body, 0, unroll=False)

    return encode


def _matmul(feats_t, w, block_n=2048):
    n = feats_t.shape[1]
    fd, od = w.shape

    def mm_kernel(f_ref, w_ref, o_ref):
        o_ref[...] = lax.dot_general(
            f_ref[...], w_ref[...], (((0,), (0,)), ((), ())),
            preferred_element_type=jnp.float32)

    return pl.pallas_call(
        mm_kernel,
        grid=(n // block_n,),
        in_specs=[
            pl.BlockSpec((fd, block_n), lambda i: (0, i)),
            pl.BlockSpec((fd, od), lambda i: (0, 0)),
        ],
        out_specs=pl.BlockSpec((block_n, od), lambda i: (i, 0)),
        out_shape=jax.ShapeDtypeStruct((n, od), jnp.float32),
    )(feats_t, w)


_encode_full = _build_encode(N_POINTS, N_LEVELS, TABLE_SIZE, chunk=256)


def kernel(inputs, tables, W):
    xyz = inputs.T
    t = jnp.transpose(tables, (2, 0, 1)).reshape(N_FEATURES, -1)
    bits = lax.bitcast_convert_type(t, jnp.int32)
    rne = bits + 0x7FFF + (lax.shift_right_logical(bits, 16) & 1)
    himask = jnp.int32(-65536)

    def pack(lo, hi):
        word = lax.shift_right_logical(rne[lo], 16) | (rne[hi] & himask)
        return lax.bitcast_convert_type(word, jnp.float32)

    feats_t = _encode_full(xyz, pack(0, 1), pack(2, 3))
    return _matmul(feats_t, W)
